# Initial kernel scaffold; baseline (speedup 1.0000x reference)
import jax
import jax.numpy as jnp
from jax.experimental import pallas as pl


def kernel(x, neighbor_ind, W, gamma, beta):
    raise NotImplementedError("write your pallas kernel here")


# speedup vs baseline: 3.9099x; 3.9099x over previous
import functools

import jax
import jax.numpy as jnp
from jax import lax
from jax.experimental import pallas as pl
from jax.experimental.pallas import tpu as pltpu
from jax.experimental.pallas import tpu_sc as plsc

D = 64
K = 16
N = 50000
NW = 32
P = 64
N_PAD = 51200
PW = N_PAD // NW
SC_ITERS = PW // P
NB = 1024
GRID = N_PAD // NB
IDX_ROWS = P * K // 128
_INV_SQRT2 = 0.7071067811865476


def _mm_body(xt_ref, wy_ref, wz_ref, y_ref, z_ref):
    xb = xt_ref[...]
    y = jnp.dot(xb, wy_ref[...], preferred_element_type=jnp.float32)
    y_ref[...] = jnp.concatenate([y, jnp.zeros_like(y)], axis=1)
    z_ref[...] = jnp.dot(xb, wz_ref[...], preferred_element_type=jnp.float32)


def _stats_body(m_ref, z_ref, s_ref):
    i = pl.program_id(0)
    t = m_ref[...][:, :D] + z_ref[...]
    rows = lax.broadcasted_iota(jnp.int32, t.shape, 0) + i * NB
    t = jnp.where(rows < N, t, 0.0)
    part = jnp.concatenate(
        [jnp.sum(t, axis=0, keepdims=True),
         jnp.sum(t * t, axis=0, keepdims=True)], axis=0)

    @pl.when(i == 0)
    def _():
        s_ref[...] = jnp.zeros_like(s_ref)

    s_ref[...] += part


def _bn_body(m_ref, z_ref, s_ref, g_ref, b_ref, o_ref):
    inv_n = 1.0 / N
    mean = s_ref[0:1, :] * inv_n
    var = s_ref[1:2, :] * inv_n - mean * mean
    scale = g_ref[...] * lax.rsqrt(var + 1e-5)
    shift = b_ref[...] - mean * scale
    t = (m_ref[...][:, :D] + z_ref[...]) * scale + shift
    o_ref[...] = t * 0.5 * (1.0 + lax.erf(t * _INV_SQRT2))


def _gather_max_body(y_hbm, nbr_hbm, out_hbm, idx_v, rows_v, m_v, sem):
    wid = lax.axis_index("s") * 2 + lax.axis_index("c")
    base = wid * PW

    def it_body(it, carry):
        row0 = pl.multiple_of(base + it * P, P)
        pltpu.sync_copy(nbr_hbm.at[pl.ds(pl.multiple_of(row0 // 8, 8), IDX_ROWS)],
                        idx_v)
        for h in range(2):
            copies = [
                pltpu.async_copy(y_hbm.at[idx_v.at[h * (IDX_ROWS // 2) + c]],
                                 rows_v.at[pl.ds(c * 128, 128)], sem)
                for c in range(IDX_ROWS // 2)
            ]
            for cp in copies:
                cp.wait()

            def p_body(p, c2, _h=h):
                r0 = p * K
                for c in range(D // 16):
                    sl = pl.ds(c * 16, 16)
                    acc = rows_v[r0, sl]
                    for j in range(1, K):
                        acc = jnp.maximum(acc, rows_v[r0 + j, sl])
                    m_v[_h * (P // 2) + p, sl] = acc
                return c2

            lax.fori_loop(0, P // 2, p_body, 0)
        pltpu.sync_copy(m_v, out_hbm.at[pl.ds(row0, P)])
        return carry

    lax.fori_loop(0, SC_ITERS, it_body, 0)


@functools.cache
def _gather_max():
    mesh = plsc.VectorSubcoreMesh(core_axis_name="c", subcore_axis_name="s")
    return pl.kernel(
        _gather_max_body,
        mesh=mesh,
        out_type=jax.ShapeDtypeStruct((N_PAD, 128), jnp.float32),
        scratch_types=[
            pltpu.VMEM((IDX_ROWS, 128), jnp.int32),
            pltpu.VMEM((P * K // 2, 128), jnp.float32),
            pltpu.VMEM((P, 128), jnp.float32),
            pltpu.SemaphoreType.DMA,
        ],
    )


def kernel(x, neighbor_ind, W, gamma, beta):
    n = x.shape[2]
    xt = jnp.pad(x[0].T, ((0, N_PAD - n), (0, 0)))
    nbr = jnp.pad(neighbor_ind[0].astype(jnp.int32),
                  ((0, N_PAD - n), (0, 0))).reshape(N_PAD // 8, 128)
    wy = W[:, :D].T
    wz = (W[:, D:] - W[:, :D]).T

    y, z = pl.pallas_call(
        _mm_body,
        grid=(GRID,),
        in_specs=[
            pl.BlockSpec((NB, D), lambda i: (i, 0)),
            pl.BlockSpec((D, D), lambda i: (0, 0)),
            pl.BlockSpec((D, D), lambda i: (0, 0)),
        ],
        out_specs=[pl.BlockSpec((NB, 128), lambda i: (i, 0)),
                   pl.BlockSpec((NB, D), lambda i: (i, 0))],
        out_shape=[jax.ShapeDtypeStruct((N_PAD, 128), jnp.float32),
                   jax.ShapeDtypeStruct((N_PAD, D), jnp.float32)],
    )(xt, wy, wz)

    m = _gather_max()(y, nbr)

    s = pl.pallas_call(
        _stats_body,
        grid=(GRID,),
        in_specs=[pl.BlockSpec((NB, 128), lambda i: (i, 0)),
                  pl.BlockSpec((NB, D), lambda i: (i, 0))],
        out_specs=pl.BlockSpec((2, D), lambda i: (0, 0)),
        out_shape=jax.ShapeDtypeStruct((2, D), jnp.float32),
    )(m, z)

    out = pl.pallas_call(
        _bn_body,
        grid=(GRID,),
        in_specs=[pl.BlockSpec((NB, 128), lambda i: (i, 0)),
                  pl.BlockSpec((NB, D), lambda i: (i, 0)),
                  pl.BlockSpec((2, D), lambda i: (0, 0)),
                  pl.BlockSpec((1, D), lambda i: (0, 0)),
                  pl.BlockSpec((1, D), lambda i: (0, 0))],
        out_specs=pl.BlockSpec((NB, D), lambda i: (i, 0)),
        out_shape=jax.ShapeDtypeStruct((N_PAD, D), jnp.float32),
    )(m, z, s, gamma.reshape(1, D), beta.reshape(1, D))

    return out[:n].T[None]


# speedup vs baseline: 4.1210x; 1.0540x over previous
import functools

import jax
import jax.numpy as jnp
from jax import lax
from jax.experimental import pallas as pl
from jax.experimental.pallas import tpu as pltpu
from jax.experimental.pallas import tpu_sc as plsc

D = 64
K = 16
N = 50000
NW = 32
N_PAD = 51200
PW = N_PAD // NW
CH = 16
CPW = PW // CH
G = CH * K // 128
IDXR = PW * K // 128
NB = 1024
GRID = N_PAD // NB
_INV_SQRT2 = 0.7071067811865476


def _mm_body(xt_ref, wy_ref, wz_ref, y_ref, z_ref):
    xb = xt_ref[...]
    y = jnp.dot(xb, wy_ref[...], preferred_element_type=jnp.float32)
    y_ref[...] = jnp.concatenate([y, jnp.zeros_like(y)], axis=1)
    z_ref[...] = jnp.dot(xb, wz_ref[...], preferred_element_type=jnp.float32)


def _stats_body(m_ref, z_ref, s_ref):
    i = pl.program_id(0)
    t = m_ref[...][:, :D] + z_ref[...]
    rows = lax.broadcasted_iota(jnp.int32, t.shape, 0) + i * NB
    t = jnp.where(rows < N, t, 0.0)
    part = jnp.concatenate(
        [jnp.sum(t, axis=0, keepdims=True),
         jnp.sum(t * t, axis=0, keepdims=True)], axis=0)

    @pl.when(i == 0)
    def _():
        s_ref[...] = jnp.zeros_like(s_ref)

    s_ref[...] += part


def _bn_body(m_ref, z_ref, s_ref, g_ref, b_ref, o_ref):
    inv_n = 1.0 / N
    mean = s_ref[0:1, :] * inv_n
    var = s_ref[1:2, :] * inv_n - mean * mean
    scale = g_ref[...] * lax.rsqrt(var + 1e-5)
    shift = b_ref[...] - mean * scale
    t = (m_ref[...][:, :D] + z_ref[...]) * scale + shift
    o_ref[...] = t * 0.5 * (1.0 + lax.erf(t * _INV_SQRT2))


def _gather_max_body(y_hbm, nbr_hbm, out_hbm, idx_all, buf0, buf1, m_v,
                     sem0, sem1):
    wid = lax.axis_index("s") * 2 + lax.axis_index("c")
    base = wid * PW
    pltpu.sync_copy(nbr_hbm.at[pl.ds(pl.multiple_of(wid * IDXR, 8), IDXR)],
                    idx_all)

    def fire(c, buf, sem):
        for q in range(G):
            pltpu.async_copy(y_hbm.at[idx_all.at[c * G + q]],
                             buf.at[pl.ds(q * 128, 128)], sem)

    def drain(buf, sem):
        for q in range(G):
            pltpu.make_async_copy(y_hbm.at[pl.ds(0, 128)],
                                  buf.at[pl.ds(q * 128, 128)], sem).wait()

    def compute(c, buf):
        def p_body(p, car):
            r0 = p * K
            for ch in range(D // 16):
                sl = pl.ds(ch * 16, 16)
                acc = buf[r0, sl]
                for j in range(1, K):
                    acc = jnp.maximum(acc, buf[r0 + j, sl])
                m_v[p, sl] = acc
            return car

        lax.fori_loop(0, CH, p_body, 0, unroll=4)
        pltpu.sync_copy(
            m_v, out_hbm.at[pl.ds(pl.multiple_of(base + c * CH, 8), CH)])

    fire(0, buf0, sem0)
    fire(1, buf1, sem1)

    def t_body(t, car):
        drain(buf0, sem0)
        compute(2 * t, buf0)

        @pl.when(t < CPW // 2 - 1)
        def _():
            fire(2 * t + 2, buf0, sem0)

        drain(buf1, sem1)
        compute(2 * t + 1, buf1)

        @pl.when(t < CPW // 2 - 1)
        def _():
            fire(2 * t + 3, buf1, sem1)

        return car

    lax.fori_loop(0, CPW // 2, t_body, 0)


@functools.cache
def _gather_max():
    mesh = plsc.VectorSubcoreMesh(core_axis_name="c", subcore_axis_name="s")
    return pl.kernel(
        _gather_max_body,
        mesh=mesh,
        out_type=jax.ShapeDtypeStruct((N_PAD, 128), jnp.float32),
        scratch_types=[
            pltpu.VMEM((IDXR, 128), jnp.int32),
            pltpu.VMEM((CH * K, 128), jnp.float32),
            pltpu.VMEM((CH * K, 128), jnp.float32),
            pltpu.VMEM((CH, 128), jnp.float32),
            pltpu.SemaphoreType.DMA,
            pltpu.SemaphoreType.DMA,
        ],
    )


def kernel(x, neighbor_ind, W, gamma, beta):
    n = x.shape[2]
    xt = jnp.pad(x[0].T, ((0, N_PAD - n), (0, 0)))
    nbr = jnp.pad(neighbor_ind[0].astype(jnp.int32),
                  ((0, N_PAD - n), (0, 0))).reshape(N_PAD // 8, 128)
    wy = W[:, :D].T
    wz = (W[:, D:] - W[:, :D]).T

    y, z = pl.pallas_call(
        _mm_body,
        grid=(GRID,),
        in_specs=[
            pl.BlockSpec((NB, D), lambda i: (i, 0)),
            pl.BlockSpec((D, D), lambda i: (0, 0)),
            pl.BlockSpec((D, D), lambda i: (0, 0)),
        ],
        out_specs=[pl.BlockSpec((NB, 128), lambda i: (i, 0)),
                   pl.BlockSpec((NB, D), lambda i: (i, 0))],
        out_shape=[jax.ShapeDtypeStruct((N_PAD, 128), jnp.float32),
                   jax.ShapeDtypeStruct((N_PAD, D), jnp.float32)],
    )(xt, wy, wz)

    m = _gather_max()(y, nbr)

    s = pl.pallas_call(
        _stats_body,
        grid=(GRID,),
        in_specs=[pl.BlockSpec((NB, 128), lambda i: (i, 0)),
                  pl.BlockSpec((NB, D), lambda i: (i, 0))],
        out_specs=pl.BlockSpec((2, D), lambda i: (0, 0)),
        out_shape=jax.ShapeDtypeStruct((2, D), jnp.float32),
    )(m, z)

    out = pl.pallas_call(
        _bn_body,
        grid=(GRID,),
        in_specs=[pl.BlockSpec((NB, 128), lambda i: (i, 0)),
                  pl.BlockSpec((NB, D), lambda i: (i, 0)),
                  pl.BlockSpec((2, D), lambda i: (0, 0)),
                  pl.BlockSpec((1, D), lambda i: (0, 0)),
                  pl.BlockSpec((1, D), lambda i: (0, 0))],
        out_specs=pl.BlockSpec((NB, D), lambda i: (i, 0)),
        out_shape=jax.ShapeDtypeStruct((N_PAD, D), jnp.float32),
    )(m, z, s, gamma.reshape(1, D), beta.reshape(1, D))

    return out[:n].T[None]


# speedup vs baseline: 4.1337x; 1.0031x over previous
import functools

import jax
import jax.numpy as jnp
from jax import lax
from jax.experimental import pallas as pl
from jax.experimental.pallas import tpu as pltpu
from jax.experimental.pallas import tpu_sc as plsc

D = 64
K = 16
N = 50000
NW = 32
N_PAD = 51200
PW = N_PAD // NW
CH = 8
CPW = PW // CH
NBUF = 5
IDXR = PW * K // 128
NB = 1024
GRID = N_PAD // NB
_INV_SQRT2 = 0.7071067811865476


def _mm_body(xt_ref, wy_ref, wz_ref, y_ref, z_ref):
    xb = xt_ref[...]
    y = jnp.dot(xb, wy_ref[...], preferred_element_type=jnp.float32)
    y_ref[...] = jnp.concatenate([y, jnp.zeros_like(y)], axis=1)
    z_ref[...] = jnp.dot(xb, wz_ref[...], preferred_element_type=jnp.float32)


def _stats_body(m_ref, z_ref, s_ref):
    i = pl.program_id(0)
    t = m_ref[...][:, :D] + z_ref[...]
    rows = lax.broadcasted_iota(jnp.int32, t.shape, 0) + i * NB
    t = jnp.where(rows < N, t, 0.0)
    part = jnp.concatenate(
        [jnp.sum(t, axis=0, keepdims=True),
         jnp.sum(t * t, axis=0, keepdims=True)], axis=0)

    @pl.when(i == 0)
    def _():
        s_ref[...] = jnp.zeros_like(s_ref)

    s_ref[...] += part


def _bn_body(m_ref, z_ref, s_ref, g_ref, b_ref, o_ref):
    inv_n = 1.0 / N
    mean = s_ref[0:1, :] * inv_n
    var = s_ref[1:2, :] * inv_n - mean * mean
    scale = g_ref[...] * lax.rsqrt(var + 1e-5)
    shift = b_ref[...] - mean * scale
    t = (m_ref[...][:, :D] + z_ref[...]) * scale + shift
    o_ref[...] = t * 0.5 * (1.0 + lax.erf(t * _INV_SQRT2))


def _gather_max_body(y_hbm, nbr_hbm, out_hbm, idx_all, *bufs_m_sems):
    bufs = bufs_m_sems[:NBUF]
    m_v = bufs_m_sems[NBUF]
    sems = bufs_m_sems[NBUF + 1:]
    wid = lax.axis_index("s") * 2 + lax.axis_index("c")
    base = wid * PW
    pltpu.sync_copy(nbr_hbm.at[pl.ds(pl.multiple_of(wid * IDXR, 8), IDXR)],
                    idx_all)

    def fire(c, b):
        pltpu.async_copy(y_hbm.at[idx_all.at[c]], bufs[b], sems[b])

    def drain(b):
        pltpu.make_async_copy(y_hbm.at[pl.ds(0, CH * K)], bufs[b],
                              sems[b]).wait()

    def compute(c, b):
        buf = bufs[b]

        def p_body(p, car):
            r0 = p * K
            for ch in range(D // 16):
                sl = pl.ds(ch * 16, 16)
                acc = buf[r0, sl]
                for j in range(1, K):
                    acc = jnp.maximum(acc, buf[r0 + j, sl])
                m_v[p, sl] = acc
            return car

        lax.fori_loop(0, CH, p_body, 0, unroll=4)
        pltpu.sync_copy(
            m_v, out_hbm.at[pl.ds(pl.multiple_of(base + c * CH, 8), CH)])

    for b in range(NBUF):
        fire(b, b)

    def t_body(t, car):
        c0 = t * NBUF
        for b in range(NBUF):
            drain(b)
            compute(c0 + b, b)

            @pl.when(c0 + b + NBUF < CPW)
            def _():
                fire(c0 + b + NBUF, b)

        return car

    lax.fori_loop(0, CPW // NBUF, t_body, 0)


@functools.cache
def _gather_max():
    mesh = plsc.VectorSubcoreMesh(core_axis_name="c", subcore_axis_name="s")
    return pl.kernel(
        _gather_max_body,
        mesh=mesh,
        out_type=jax.ShapeDtypeStruct((N_PAD, 128), jnp.float32),
        scratch_types=(
            [pltpu.VMEM((IDXR, 128), jnp.int32)]
            + [pltpu.VMEM((CH * K, 128), jnp.float32)
               for _ in range(NBUF)]
            + [pltpu.VMEM((CH, 128), jnp.float32)]
            + [pltpu.SemaphoreType.DMA for _ in range(NBUF)]
        ),
    )


def kernel(x, neighbor_ind, W, gamma, beta):
    n = x.shape[2]
    xt = jnp.pad(x[0].T, ((0, N_PAD - n), (0, 0)))
    nbr = jnp.pad(neighbor_ind[0].astype(jnp.int32),
                  ((0, N_PAD - n), (0, 0))).reshape(N_PAD // 8, 128)
    wy = W[:, :D].T
    wz = (W[:, D:] - W[:, :D]).T

    y, z = pl.pallas_call(
        _mm_body,
        grid=(GRID,),
        in_specs=[
            pl.BlockSpec((NB, D), lambda i: (i, 0)),
            pl.BlockSpec((D, D), lambda i: (0, 0)),
            pl.BlockSpec((D, D), lambda i: (0, 0)),
        ],
        out_specs=[pl.BlockSpec((NB, 128), lambda i: (i, 0)),
                   pl.BlockSpec((NB, D), lambda i: (i, 0))],
        out_shape=[jax.ShapeDtypeStruct((N_PAD, 128), jnp.float32),
                   jax.ShapeDtypeStruct((N_PAD, D), jnp.float32)],
    )(xt, wy, wz)

    m = _gather_max()(y, nbr)

    s = pl.pallas_call(
        _stats_body,
        grid=(GRID,),
        in_specs=[pl.BlockSpec((NB, 128), lambda i: (i, 0)),
                  pl.BlockSpec((NB, D), lambda i: (i, 0))],
        out_specs=pl.BlockSpec((2, D), lambda i: (0, 0)),
        out_shape=jax.ShapeDtypeStruct((2, D), jnp.float32),
    )(m, z)

    out = pl.pallas_call(
        _bn_body,
        grid=(GRID,),
        in_specs=[pl.BlockSpec((NB, 128), lambda i: (i, 0)),
                  pl.BlockSpec((NB, D), lambda i: (i, 0)),
                  pl.BlockSpec((2, D), lambda i: (0, 0)),
                  pl.BlockSpec((1, D), lambda i: (0, 0)),
                  pl.BlockSpec((1, D), lambda i: (0, 0))],
        out_specs=pl.BlockSpec((NB, D), lambda i: (i, 0)),
        out_shape=jax.ShapeDtypeStruct((N_PAD, D), jnp.float32),
    )(m, z, s, gamma.reshape(1, D), beta.reshape(1, D))

    return out[:n].T[None]
